# baseline (speedup 1.0000x reference)
import functools

import jax
import jax.numpy as jnp
from jax import lax
from jax.experimental import pallas as pl
from jax.experimental.pallas import tpu as pltpu
from jax.experimental.pallas import tpu_sc as plsc

UNUM = 6000
INUM = 4000
D = 128
L = 16
NC = 2
NS = 16
NW = NC * NS
CH = 128

NB_GCN = UNUM + INUM
OFF_DU0 = 10000
OFF_DN0 = 16000
OFF_DU1 = 22000
OFF_DN1 = 28000
OFF_DI0 = 34000
OFF_DNI0 = 38000
OFF_DI1 = 42000
OFF_DNI1 = 46000
SAC_BIN = 50000
NBINS = 50048

NPAD_G = 10112
NPAD_U = 12032
NPAD_I = 8064
EPAD_G = 327680
EPAD_U = 393216
EPAD_I = 262144
EPAD_D = 1605632

_MESH = plsc.VectorSubcoreMesh(core_axis_name="c", subcore_axis_name="s",
                               num_cores=NC, num_subcores=NS)


def _zero_shared(zin, acc, sid, rps, zrows):
    start = sid * rps
    k = 0
    off = 0
    while off < rps:
        sz = min(zrows, rps - off)
        pltpu.sync_copy(zin.at[pl.ds(0, sz)], acc.at[pl.ds(start + off, sz)])
        off += sz
        k += 1


def _make_spmm(npad, e_pad):
    nbuf = 2
    npw = e_pad // (NW * CH)
    nit = npw // nbuf
    rps = npad // NS

    @functools.partial(
        pl.kernel,
        out_type=jax.ShapeDtypeStruct((NC * npad, D), jnp.float32),
        mesh=_MESH,
        scratch_types=[
            pltpu.VMEM((CH,), jnp.int32),
            pltpu.VMEM((CH,), jnp.int32),
            pltpu.VMEM((CH,), jnp.int32),
            pltpu.VMEM((CH,), jnp.int32),
        ] + [pltpu.VMEM((CH, D), jnp.float32) for _ in range(nbuf)] + [
            pltpu.VMEM_SHARED((npad, D), jnp.float32),
        ] + [pltpu.SemaphoreType.DMA for _ in range(nbuf)],
    )
    def spmm(h_hbm, src_hbm, dst_hbm, zin_hbm, out_hbm, sa, sb, da, db,
             r0, r1, acc, g0, g1):
        rows = [r0, r1]
        gsem = [g0, g1]
        sidx = [sa, sb]
        didx = [da, db]
        cid = lax.axis_index("c")
        sid = lax.axis_index("s")
        wid = sid * NC + cid
        _zero_shared(zin_hbm, acc, sid, rps, 512)
        plsc.subcore_barrier()

        base_chunk = wid * npw

        def body(g, _):
            e0 = (base_chunk + g * nbuf) * CH
            for b in range(nbuf):
                pltpu.sync_copy(src_hbm.at[pl.ds(e0 + b * CH, CH)], sidx[b])
                pltpu.sync_copy(dst_hbm.at[pl.ds(e0 + b * CH, CH)], didx[b])
                pltpu.async_copy(h_hbm.at[sidx[b]], rows[b], gsem[b]).wait()
                pltpu.sync_copy(rows[b], acc.at[didx[b]], add=True)
            return 0

        lax.fori_loop(0, nit, body, 0)
        plsc.subcore_barrier()
        start = sid * rps
        off = 0
        while off < rps:
            sz = min(512, rps - off)
            pltpu.sync_copy(acc.at[pl.ds(start + off, sz)],
                            out_hbm.at[pl.ds(cid * npad + start + off, sz)])
            off += sz

    return spmm


_spmm_g = _make_spmm(NPAD_G, EPAD_G)
_spmm_u = _make_spmm(NPAD_U, EPAD_U)
_spmm_i = _make_spmm(NPAD_I, EPAD_I)

_DEG_NPW = EPAD_D // (NW * CH)
_DEG_RPS = NBINS // NS


@functools.partial(
    pl.kernel,
    out_type=jax.ShapeDtypeStruct((NW, NBINS), jnp.float32),
    mesh=_MESH,
    compiler_params=pltpu.CompilerParams(needs_layout_passes=False),
    scratch_types=[
        pltpu.VMEM((1024,), jnp.int32),
        pltpu.VMEM((NBINS,), jnp.float32),
    ],
)
def _deg_kernel(idx_hbm, zin_hbm, out_hbm, didx, hist):
    cid = lax.axis_index("c")
    sid = lax.axis_index("s")
    wid = sid * NC + cid
    pltpu.sync_copy(zin_hbm, hist)
    epw = EPAD_D // NW
    nit = epw // 1024
    base = wid * epw
    ones16 = jnp.ones((L,), jnp.float32)

    def body(k, _):
        pltpu.sync_copy(idx_hbm.at[pl.ds(base + k * 1024, 1024)], didx)

        def inner(j, _):
            iv = didx[pl.ds(j * L, L)]
            plsc.addupdate_scatter(hist, [iv], ones16)
            return 0

        lax.fori_loop(0, 64, inner, 0)
        return 0

    lax.fori_loop(0, nit, body, 0)
    pltpu.sync_copy(hist, out_hbm.at[wid])


@functools.partial(
    pl.kernel,
    out_type=(jax.ShapeDtypeStruct((4096, D), jnp.float32),
              jax.ShapeDtypeStruct((4096, D), jnp.float32),
              jax.ShapeDtypeStruct((4096, D), jnp.float32)),
    mesh=_MESH,
    scratch_types=[
        pltpu.VMEM((CH,), jnp.int32),
        pltpu.VMEM((CH, D), jnp.float32),
        pltpu.SemaphoreType.DMA,
    ],
)
def _gather_kernel(uemb_hbm, iemb_hbm, uidx_hbm, iidx_hbm, cidx_hbm,
                   ue_hbm, ie_hbm, ne_hbm, idx_v, rows, sem):
    cid = lax.axis_index("c")
    sid = lax.axis_index("s")
    wid = sid * NC + cid
    base = wid * CH
    pltpu.sync_copy(uidx_hbm.at[pl.ds(base, CH)], idx_v)
    pltpu.async_copy(uemb_hbm.at[idx_v], rows, sem).wait()
    pltpu.sync_copy(rows, ue_hbm.at[pl.ds(base, CH)])
    pltpu.sync_copy(iidx_hbm.at[pl.ds(base, CH)], idx_v)
    pltpu.async_copy(iemb_hbm.at[idx_v], rows, sem).wait()
    pltpu.sync_copy(rows, ie_hbm.at[pl.ds(base, CH)])
    pltpu.sync_copy(cidx_hbm.at[pl.ds(base, CH)], idx_v)
    pltpu.async_copy(iemb_hbm.at[idx_v], rows, sem).wait()
    pltpu.sync_copy(rows, ne_hbm.at[pl.ds(base, CH)])


def _dinv_body(degp_ref, dinvg_ref, svg2_ref, dnu_ref, dni_ref, duu_ref,
               dui_ref):
    deg = jnp.sum(degp_ref[...], axis=0)
    dg = deg[0:NB_GCN]
    dinv_g = jnp.where(dg > 0, lax.rsqrt(dg), 0.0)

    def cl(o, n):
        return lax.rsqrt(jnp.maximum(deg[o:o + n], 1.0))

    dinvg_ref[...] = dinv_g
    svg2_ref[...] = dinv_g * dinv_g
    dnu_ref[...] = jnp.stack([cl(OFF_DN0, UNUM), cl(OFF_DN1, UNUM)], axis=0)
    dni_ref[...] = jnp.stack([cl(OFF_DNI0, INUM), cl(OFF_DNI1, INUM)], axis=0)
    duu_ref[...] = jnp.stack([cl(OFF_DU0, UNUM), cl(OFF_DU1, UNUM)], axis=0)
    dui_ref[...] = jnp.stack([cl(OFF_DI0, INUM), cl(OFF_DI1, INUM)], axis=0)


def _hg0_body(fu_ref, fi_ref, dinvg_ref, hg0_ref):
    dinv_g = dinvg_ref[...]
    hg0_ref[0:UNUM, :] = fu_ref[...] * dinv_g[0:UNUM, None]
    hg0_ref[UNUM:NB_GCN, :] = fi_ref[...] * dinv_g[UNUM:NB_GCN, None]


def _make_prescale_body(n):
    def body(f_ref, du_ref, h0_ref):
        du = du_ref[...]
        f = f_ref[...]
        h0_ref[0:n, :] = f * du[0][:, None]
        h0_ref[n:2 * n, :] = f * du[1][:, None]
    return body


def _gcn_scale_body(pg_ref, svg2_ref, hg1_ref):
    agg = pg_ref[0:NB_GCN, :] + pg_ref[NPAD_G:NPAD_G + NB_GCN, :]
    hg1_ref[...] = agg * svg2_ref[...][:, None]


def _attn(e0, e1, W1, b1, W2):
    s0 = jnp.mean(jnp.dot(jnp.tanh(jnp.dot(e0, W1) + b1[None, :]), W2,
                          preferred_element_type=jnp.float32))
    s1 = jnp.mean(jnp.dot(jnp.tanh(jnp.dot(e1, W1) + b1[None, :]), W2,
                          preferred_element_type=jnp.float32))
    m = jnp.maximum(s0, s1)
    a0 = jnp.exp(s0 - m)
    a1 = jnp.exp(s1 - m)
    return (a0 * e0 + a1 * e1) / (a0 + a1)


def _make_han_body(n, npad, prescale):
    def body(*refs):
        if prescale:
            (p_ref, dn_ref, du_ref, w1_ref, b1_ref, w2_ref, out_ref) = refs
        else:
            (p_ref, dn_ref, w1_ref, b1_ref, w2_ref, out_ref) = refs
        dn = dn_ref[...]
        e0 = (p_ref[0:n, :] + p_ref[npad:npad + n, :]) * dn[0][:, None]
        e1 = (p_ref[n:2 * n, :] + p_ref[npad + n:npad + 2 * n, :]) * dn[1][:, None]
        h = _attn(e0, e1, w1_ref[...], b1_ref[...], w2_ref[...])
        if prescale:
            du = du_ref[...]
            out_ref[...] = jnp.concatenate(
                [h * du[0][:, None], h * du[1][:, None]], axis=0)
        else:
            out_ref[...] = h
    return body


def _final_mix_body(pg_ref, dinvg_ref, hu_ref, hi_ref, ue_ref, ie_ref):
    ui2 = (pg_ref[0:NB_GCN, :] + pg_ref[NPAD_G:NPAD_G + NB_GCN, :]) \
        * dinvg_ref[...][:, None]
    ue_ref[...] = 0.5 * hu_ref[...] + 0.5 * ui2[0:UNUM, :]
    ie_ref[...] = 0.5 * hi_ref[...] + 0.5 * ui2[UNUM:NB_GCN, :]


def _f32(shape):
    return jax.ShapeDtypeStruct(shape, jnp.float32)


def _pad_i32(x, n, fill):
    return jnp.concatenate(
        [x.astype(jnp.int32),
         jnp.full((n - x.shape[0],), fill, dtype=jnp.int32)])


def kernel(feat_user, feat_item, ui_src, ui_dst, mp_user_src, mp_user_dst,
           mp_item_src, mp_item_dst, W1_u, b1_u, W2_u, W1_i, b1_i, W2_i,
           user_idx, item_idx, neg_item_idx):
    ui_src = ui_src.astype(jnp.int32)
    ui_dst = ui_dst.astype(jnp.int32)
    mp_user_src = mp_user_src.astype(jnp.int32)
    mp_user_dst = mp_user_dst.astype(jnp.int32)
    mp_item_src = mp_item_src.astype(jnp.int32)
    mp_item_dst = mp_item_dst.astype(jnp.int32)

    g_src = _pad_i32(jnp.concatenate([ui_dst + UNUM, ui_src]), EPAD_G, 0)
    g_dst = _pad_i32(jnp.concatenate([ui_src, ui_dst + UNUM]), EPAD_G,
                     NB_GCN)
    u_src = _pad_i32(jnp.concatenate([mp_user_src[0], mp_user_src[1] + UNUM]),
                     EPAD_U, 0)
    u_dst = _pad_i32(jnp.concatenate([mp_user_dst[0], mp_user_dst[1] + UNUM]),
                     EPAD_U, 2 * UNUM)
    i_src = _pad_i32(jnp.concatenate([mp_item_src[0], mp_item_src[1] + INUM]),
                     EPAD_I, 0)
    i_dst = _pad_i32(jnp.concatenate([mp_item_dst[0], mp_item_dst[1] + INUM]),
                     EPAD_I, 2 * INUM)
    d_idx = _pad_i32(jnp.concatenate([
        ui_src, ui_dst + UNUM,
        mp_user_src[0] + OFF_DU0, mp_user_dst[0] + OFF_DN0,
        mp_user_src[1] + OFF_DU1, mp_user_dst[1] + OFF_DN1,
        mp_item_src[0] + OFF_DI0, mp_item_dst[0] + OFF_DNI0,
        mp_item_src[1] + OFF_DI1, mp_item_dst[1] + OFF_DNI1,
    ]), EPAD_D, SAC_BIN)

    zin = jnp.zeros((512, D), jnp.float32)
    zin16 = jnp.zeros((2048, L), jnp.float32)
    e1row = jnp.zeros((CH, L), jnp.float32).at[:, 0].set(1.0)

    degp = _deg_kernel(d_idx, jnp.zeros((NBINS,), jnp.float32))

    dinvg, svg2, dnu, dni, duu, dui = pl.pallas_call(
        _dinv_body,
        out_shape=(_f32((NB_GCN,)), _f32((NB_GCN,)), _f32((2, UNUM)),
                   _f32((2, INUM)), _f32((2, UNUM)), _f32((2, INUM))),
    )(degp)
    hg = pl.pallas_call(_hg0_body, out_shape=_f32((NB_GCN, D)))(
        feat_user, feat_item, dinvg)
    hu = pl.pallas_call(_make_prescale_body(UNUM),
                        out_shape=_f32((2 * UNUM, D)))(feat_user, duu)
    hi = pl.pallas_call(_make_prescale_body(INUM),
                        out_shape=_f32((2 * INUM, D)))(feat_item, dui)

    han_u_pre = _make_han_body(UNUM, NPAD_U, True)
    han_i_pre = _make_han_body(INUM, NPAD_I, True)
    han_u_fin = _make_han_body(UNUM, NPAD_U, False)
    han_i_fin = _make_han_body(INUM, NPAD_I, False)

    pg = _spmm_g(hg, g_src, g_dst, zin)
    pu = _spmm_u(hu, u_src, u_dst, zin)
    pi = _spmm_i(hi, i_src, i_dst, zin)
    hg = pl.pallas_call(_gcn_scale_body, out_shape=_f32((NB_GCN, D)))(pg, svg2)
    hu = pl.pallas_call(han_u_pre, out_shape=_f32((2 * UNUM, D)))(
        pu, dnu, duu, W1_u, b1_u, W2_u)
    hi = pl.pallas_call(han_i_pre, out_shape=_f32((2 * INUM, D)))(
        pi, dni, dui, W1_i, b1_i, W2_i)

    pg = _spmm_g(hg, g_src, g_dst, zin)
    pu = _spmm_u(hu, u_src, u_dst, zin)
    pi = _spmm_i(hi, i_src, i_dst, zin)
    hu2 = pl.pallas_call(han_u_fin, out_shape=_f32((UNUM, D)))(
        pu, dnu, W1_u, b1_u, W2_u)
    hi2 = pl.pallas_call(han_i_fin, out_shape=_f32((INUM, D)))(
        pi, dni, W1_i, b1_i, W2_i)
    user_emb, item_emb = pl.pallas_call(
        _final_mix_body,
        out_shape=(_f32((UNUM, D)), _f32((INUM, D))),
    )(pg, dinvg, hu2, hi2)

    item_idx = item_idx.astype(jnp.int32)
    cidx = item_idx[neg_item_idx.astype(jnp.int32)]
    ue, ie, ne = _gather_kernel(user_emb, item_emb,
                                user_idx.astype(jnp.int32), item_idx, cidx)
    return (ue, ie, ne)


# speedup vs baseline: 1.7167x; 1.7167x over previous
import functools

import jax
import jax.numpy as jnp
from jax import lax
from jax.experimental import pallas as pl
from jax.experimental.pallas import tpu as pltpu
from jax.experimental.pallas import tpu_sc as plsc

UNUM = 6000
INUM = 4000
D = 128
L = 16
NC = 2
NS = 16
NW = NC * NS
CH = 128

NB_GCN = UNUM + INUM
OFF_DU0 = 10000
OFF_DN0 = 16000
OFF_DU1 = 22000
OFF_DN1 = 28000
OFF_DI0 = 34000
OFF_DNI0 = 38000
OFF_DI1 = 42000
OFF_DNI1 = 46000
SAC_BIN = 50000
NBINS = 50048

NPAD_G = 10112
NPAD_U = 12032
NPAD_I = 8064
EPAD_G = 323584
EPAD_U = 385024
EPAD_I = 258048
EPAD_D = 1605632

_MESH = plsc.VectorSubcoreMesh(core_axis_name="c", subcore_axis_name="s",
                               num_cores=NC, num_subcores=NS)


def _zero_shared(zin, acc, sid, rps, zrows):
    start = sid * rps
    k = 0
    off = 0
    while off < rps:
        sz = min(zrows, rps - off)
        pltpu.sync_copy(zin.at[pl.ds(0, sz)], acc.at[pl.ds(start + off, sz)])
        off += sz
        k += 1


def _make_spmm(npad, e_pad):
    npw = e_pad // (NW * CH)
    rps = npad // NS

    @functools.partial(
        pl.kernel,
        out_type=jax.ShapeDtypeStruct((NC * npad, D), jnp.float32),
        mesh=_MESH,
        scratch_types=[
            pltpu.VMEM((CH,), jnp.int32),
            pltpu.VMEM((CH,), jnp.int32),
            pltpu.VMEM((CH, D), jnp.float32),
            pltpu.VMEM_SHARED((npad, D), jnp.float32),
            pltpu.SemaphoreType.DMA,
        ],
    )
    def spmm(h_hbm, src_hbm, dst_hbm, zin_hbm, out_hbm, sidx, didx, rows, acc,
             sem):
        cid = lax.axis_index("c")
        sid = lax.axis_index("s")
        wid = sid * NC + cid
        _zero_shared(zin_hbm, acc, sid, rps, 512)
        plsc.subcore_barrier()

        base_chunk = wid * npw

        def body(k, _):
            e0 = (base_chunk + k) * CH
            pltpu.sync_copy(src_hbm.at[pl.ds(e0, CH)], sidx)
            pltpu.sync_copy(dst_hbm.at[pl.ds(e0, CH)], didx)
            pltpu.async_copy(h_hbm.at[sidx], rows, sem).wait()
            pltpu.sync_copy(rows, acc.at[didx], add=True)
            return 0

        lax.fori_loop(0, npw, body, 0)
        plsc.subcore_barrier()
        start = sid * rps
        off = 0
        while off < rps:
            sz = min(512, rps - off)
            pltpu.sync_copy(acc.at[pl.ds(start + off, sz)],
                            out_hbm.at[pl.ds(cid * npad + start + off, sz)])
            off += sz

    return spmm


_spmm_g = _make_spmm(NPAD_G, EPAD_G)
_spmm_u = _make_spmm(NPAD_U, EPAD_U)
_spmm_i = _make_spmm(NPAD_I, EPAD_I)

_DEG_NPW = EPAD_D // (NW * CH)
_DEG_RPS = NBINS // NS


@functools.partial(
    pl.kernel,
    out_type=jax.ShapeDtypeStruct((NW, NBINS), jnp.float32),
    mesh=_MESH,
    compiler_params=pltpu.CompilerParams(needs_layout_passes=False),
    scratch_types=[
        pltpu.VMEM((1024,), jnp.int32),
        pltpu.VMEM((NBINS,), jnp.float32),
    ],
)
def _deg_kernel(idx_hbm, zin_hbm, out_hbm, didx, hist):
    cid = lax.axis_index("c")
    sid = lax.axis_index("s")
    wid = sid * NC + cid
    pltpu.sync_copy(zin_hbm, hist)
    epw = EPAD_D // NW
    base = wid * epw
    ones16 = jnp.ones((L,), jnp.float32)

    def body(k, _):
        pltpu.sync_copy(idx_hbm.at[pl.ds(base + k * 1024, 1024)], didx)

        def inner(j, _):
            iv = didx[pl.ds(j * L, L)]
            plsc.addupdate_scatter(hist, [iv], ones16)
            return 0

        lax.fori_loop(0, 64, inner, 0)
        return 0

    lax.fori_loop(0, epw // 1024, body, 0)
    pltpu.sync_copy(hist, out_hbm.at[wid])


@functools.partial(
    pl.kernel,
    out_type=(jax.ShapeDtypeStruct((4096, D), jnp.float32),
              jax.ShapeDtypeStruct((4096, D), jnp.float32),
              jax.ShapeDtypeStruct((4096, D), jnp.float32)),
    mesh=_MESH,
    scratch_types=[
        pltpu.VMEM((CH,), jnp.int32),
        pltpu.VMEM((CH, D), jnp.float32),
        pltpu.SemaphoreType.DMA,
    ],
)
def _gather_kernel(uemb_hbm, iemb_hbm, uidx_hbm, iidx_hbm, cidx_hbm,
                   ue_hbm, ie_hbm, ne_hbm, idx_v, rows, sem):
    cid = lax.axis_index("c")
    sid = lax.axis_index("s")
    wid = sid * NC + cid
    base = wid * CH
    pltpu.sync_copy(uidx_hbm.at[pl.ds(base, CH)], idx_v)
    pltpu.async_copy(uemb_hbm.at[idx_v], rows, sem).wait()
    pltpu.sync_copy(rows, ue_hbm.at[pl.ds(base, CH)])
    pltpu.sync_copy(iidx_hbm.at[pl.ds(base, CH)], idx_v)
    pltpu.async_copy(iemb_hbm.at[idx_v], rows, sem).wait()
    pltpu.sync_copy(rows, ie_hbm.at[pl.ds(base, CH)])
    pltpu.sync_copy(cidx_hbm.at[pl.ds(base, CH)], idx_v)
    pltpu.async_copy(iemb_hbm.at[idx_v], rows, sem).wait()
    pltpu.sync_copy(rows, ne_hbm.at[pl.ds(base, CH)])


def _dinv_body(degp_ref, dinvg_ref, svg2_ref, dnu_ref, dni_ref, duu_ref,
               dui_ref):
    deg = jnp.sum(degp_ref[...], axis=0)
    dg = deg[0:NB_GCN]
    dinv_g = jnp.where(dg > 0, lax.rsqrt(dg), 0.0)

    def cl(o, n):
        return lax.rsqrt(jnp.maximum(deg[o:o + n], 1.0))

    dinvg_ref[...] = dinv_g
    svg2_ref[...] = dinv_g * dinv_g
    dnu_ref[...] = jnp.stack([cl(OFF_DN0, UNUM), cl(OFF_DN1, UNUM)], axis=0)
    dni_ref[...] = jnp.stack([cl(OFF_DNI0, INUM), cl(OFF_DNI1, INUM)], axis=0)
    duu_ref[...] = jnp.stack([cl(OFF_DU0, UNUM), cl(OFF_DU1, UNUM)], axis=0)
    dui_ref[...] = jnp.stack([cl(OFF_DI0, INUM), cl(OFF_DI1, INUM)], axis=0)


def _hg0_body(fu_ref, fi_ref, dinvg_ref, hg0_ref):
    dinv_g = dinvg_ref[...]
    hg0_ref[0:UNUM, :] = fu_ref[...] * dinv_g[0:UNUM, None]
    hg0_ref[UNUM:NB_GCN, :] = fi_ref[...] * dinv_g[UNUM:NB_GCN, None]


def _make_prescale_body(n):
    def body(f_ref, du_ref, h0_ref):
        du = du_ref[...]
        f = f_ref[...]
        h0_ref[0:n, :] = f * du[0][:, None]
        h0_ref[n:2 * n, :] = f * du[1][:, None]
    return body


def _gcn_scale_body(pg_ref, svg2_ref, hg1_ref):
    agg = pg_ref[0:NB_GCN, :] + pg_ref[NPAD_G:NPAD_G + NB_GCN, :]
    hg1_ref[...] = agg * svg2_ref[...][:, None]


def _attn(e0, e1, W1, b1, W2):
    s0 = jnp.mean(jnp.dot(jnp.tanh(jnp.dot(e0, W1) + b1[None, :]), W2,
                          preferred_element_type=jnp.float32))
    s1 = jnp.mean(jnp.dot(jnp.tanh(jnp.dot(e1, W1) + b1[None, :]), W2,
                          preferred_element_type=jnp.float32))
    m = jnp.maximum(s0, s1)
    a0 = jnp.exp(s0 - m)
    a1 = jnp.exp(s1 - m)
    return (a0 * e0 + a1 * e1) / (a0 + a1)


def _make_han_body(n, npad, prescale):
    def body(*refs):
        if prescale:
            (p_ref, dn_ref, du_ref, w1_ref, b1_ref, w2_ref, out_ref) = refs
        else:
            (p_ref, dn_ref, w1_ref, b1_ref, w2_ref, out_ref) = refs
        dn = dn_ref[...]
        e0 = (p_ref[0:n, :] + p_ref[npad:npad + n, :]) * dn[0][:, None]
        e1 = (p_ref[n:2 * n, :] + p_ref[npad + n:npad + 2 * n, :]) * dn[1][:, None]
        h = _attn(e0, e1, w1_ref[...], b1_ref[...], w2_ref[...])
        if prescale:
            du = du_ref[...]
            out_ref[...] = jnp.concatenate(
                [h * du[0][:, None], h * du[1][:, None]], axis=0)
        else:
            out_ref[...] = h
    return body


def _final_mix_body(pg_ref, dinvg_ref, hu_ref, hi_ref, ue_ref, ie_ref):
    ui2 = (pg_ref[0:NB_GCN, :] + pg_ref[NPAD_G:NPAD_G + NB_GCN, :]) \
        * dinvg_ref[...][:, None]
    ue_ref[...] = 0.5 * hu_ref[...] + 0.5 * ui2[0:UNUM, :]
    ie_ref[...] = 0.5 * hi_ref[...] + 0.5 * ui2[UNUM:NB_GCN, :]


def _f32(shape):
    return jax.ShapeDtypeStruct(shape, jnp.float32)


def _pad_i32(x, n, fill):
    return jnp.concatenate(
        [x.astype(jnp.int32),
         jnp.full((n - x.shape[0],), fill, dtype=jnp.int32)])


def kernel(feat_user, feat_item, ui_src, ui_dst, mp_user_src, mp_user_dst,
           mp_item_src, mp_item_dst, W1_u, b1_u, W2_u, W1_i, b1_i, W2_i,
           user_idx, item_idx, neg_item_idx):
    ui_src = ui_src.astype(jnp.int32)
    ui_dst = ui_dst.astype(jnp.int32)
    mp_user_src = mp_user_src.astype(jnp.int32)
    mp_user_dst = mp_user_dst.astype(jnp.int32)
    mp_item_src = mp_item_src.astype(jnp.int32)
    mp_item_dst = mp_item_dst.astype(jnp.int32)

    g_src = _pad_i32(jnp.concatenate([ui_dst + UNUM, ui_src]), EPAD_G, 0)
    g_dst = _pad_i32(jnp.concatenate([ui_src, ui_dst + UNUM]), EPAD_G, NB_GCN)
    u_src = _pad_i32(jnp.concatenate([mp_user_src[0], mp_user_src[1] + UNUM]),
                     EPAD_U, 0)
    u_dst = _pad_i32(jnp.concatenate([mp_user_dst[0], mp_user_dst[1] + UNUM]),
                     EPAD_U, 2 * UNUM)
    i_src = _pad_i32(jnp.concatenate([mp_item_src[0], mp_item_src[1] + INUM]),
                     EPAD_I, 0)
    i_dst = _pad_i32(jnp.concatenate([mp_item_dst[0], mp_item_dst[1] + INUM]),
                     EPAD_I, 2 * INUM)
    d_idx = _pad_i32(jnp.concatenate([
        ui_src, ui_dst + UNUM,
        mp_user_src[0] + OFF_DU0, mp_user_dst[0] + OFF_DN0,
        mp_user_src[1] + OFF_DU1, mp_user_dst[1] + OFF_DN1,
        mp_item_src[0] + OFF_DI0, mp_item_dst[0] + OFF_DNI0,
        mp_item_src[1] + OFF_DI1, mp_item_dst[1] + OFF_DNI1,
    ]), EPAD_D, SAC_BIN)

    zin = jnp.zeros((512, D), jnp.float32)
    zin16 = jnp.zeros((2048, L), jnp.float32)
    e1row = jnp.zeros((CH, L), jnp.float32).at[:, 0].set(1.0)

    degp = _deg_kernel(d_idx, jnp.zeros((NBINS,), jnp.float32))

    dinvg, svg2, dnu, dni, duu, dui = pl.pallas_call(
        _dinv_body,
        out_shape=(_f32((NB_GCN,)), _f32((NB_GCN,)), _f32((2, UNUM)),
                   _f32((2, INUM)), _f32((2, UNUM)), _f32((2, INUM))),
    )(degp)
    hg = pl.pallas_call(_hg0_body, out_shape=_f32((NB_GCN, D)))(
        feat_user, feat_item, dinvg)
    hu = pl.pallas_call(_make_prescale_body(UNUM),
                        out_shape=_f32((2 * UNUM, D)))(feat_user, duu)
    hi = pl.pallas_call(_make_prescale_body(INUM),
                        out_shape=_f32((2 * INUM, D)))(feat_item, dui)

    han_u_pre = _make_han_body(UNUM, NPAD_U, True)
    han_i_pre = _make_han_body(INUM, NPAD_I, True)
    han_u_fin = _make_han_body(UNUM, NPAD_U, False)
    han_i_fin = _make_han_body(INUM, NPAD_I, False)

    pg = _spmm_g(hg, g_src, g_dst, zin)
    pu = _spmm_u(hu, u_src, u_dst, zin)
    pi = _spmm_i(hi, i_src, i_dst, zin)
    hg = pl.pallas_call(_gcn_scale_body, out_shape=_f32((NB_GCN, D)))(pg, svg2)
    hu = pl.pallas_call(han_u_pre, out_shape=_f32((2 * UNUM, D)))(
        pu, dnu, duu, W1_u, b1_u, W2_u)
    hi = pl.pallas_call(han_i_pre, out_shape=_f32((2 * INUM, D)))(
        pi, dni, dui, W1_i, b1_i, W2_i)

    pg = _spmm_g(hg, g_src, g_dst, zin)
    pu = _spmm_u(hu, u_src, u_dst, zin)
    pi = _spmm_i(hi, i_src, i_dst, zin)
    hu2 = pl.pallas_call(han_u_fin, out_shape=_f32((UNUM, D)))(
        pu, dnu, W1_u, b1_u, W2_u)
    hi2 = pl.pallas_call(han_i_fin, out_shape=_f32((INUM, D)))(
        pi, dni, W1_i, b1_i, W2_i)
    user_emb, item_emb = pl.pallas_call(
        _final_mix_body,
        out_shape=(_f32((UNUM, D)), _f32((INUM, D))),
    )(pg, dinvg, hu2, hi2)

    item_idx = item_idx.astype(jnp.int32)
    cidx = item_idx[neg_item_idx.astype(jnp.int32)]
    ue, ie, ne = _gather_kernel(user_emb, item_emb,
                                user_idx.astype(jnp.int32), item_idx, cidx)
    return (ue, ie, ne)
